# Initial kernel scaffold; baseline (speedup 1.0000x reference)
import jax
import jax.numpy as jnp
from jax.experimental import pallas as pl


def kernel(x, edge_index, batch, feature, W1, b1, W2, b2, W3, b3, Wg, bg, Wf1, bf1, Wf2, bf2):
    raise NotImplementedError("write your pallas kernel here")


# speedup vs baseline: 1.0442x; 1.0442x over previous
import functools

import jax
import jax.numpy as jnp
from jax.experimental import pallas as pl

_N_BLOCK = 2000


def _mm_body(x_ref, w_ref, o_ref):
    o_ref[...] = jnp.dot(x_ref[...], w_ref[...],
                         preferred_element_type=jnp.float32)


@functools.partial(jax.jit, static_argnames=())
def _matmul(x, w):
    n, d = x.shape
    do = w.shape[1]
    grid = (n // _N_BLOCK,)
    return pl.pallas_call(
        _mm_body,
        grid=grid,
        in_specs=[
            pl.BlockSpec((_N_BLOCK, d), lambda i: (i, 0)),
            pl.BlockSpec((d, do), lambda i: (0, 0)),
        ],
        out_specs=pl.BlockSpec((_N_BLOCK, do), lambda i: (i, 0)),
        out_shape=jax.ShapeDtypeStruct((n, do), jnp.float32),
    )(x, w)


def _head_body(x1_ref, wg_ref, bg_ref, ft_ref, wf1_ref, bf1_ref,
               wf2_ref, bf2_ref, o_ref):
    g = jnp.maximum(
        jnp.dot(x1_ref[...], wg_ref[...],
                preferred_element_type=jnp.float32) + bg_ref[...], 0.0)
    h = jnp.maximum(
        jnp.dot(ft_ref[...], wf1_ref[...],
                preferred_element_type=jnp.float32) + bf1_ref[...], 0.0)
    o_ref[...] = g + jnp.dot(h, wf2_ref[...],
                             preferred_element_type=jnp.float32) + bf2_ref[...]


def _head(x1, wg, bg, feature, wf1, bf1, wf2, bf2):
    b = x1.shape[0]
    return pl.pallas_call(
        _head_body,
        out_shape=jax.ShapeDtypeStruct((b, 1), jnp.float32),
    )(x1, wg, bg.reshape(1, 1), feature, wf1, bf1.reshape(1, 128),
      wf2, bf2.reshape(1, 1))


def kernel(x, edge_index, batch, feature, W1, b1, W2, b2, W3, b3,
           Wg, bg, Wf1, bf1, Wf2, bf2):
    n = x.shape[0]
    bsz = feature.shape[0]
    src = edge_index[0]
    dst = edge_index[1]

    deg = jnp.zeros((n,), jnp.float32).at[dst].add(1.0) + 1.0
    dinv = jax.lax.rsqrt(deg)
    norm_e = (dinv[src] * dinv[dst])[:, None]
    dinv2 = (dinv * dinv)[:, None]

    def conv(h, w, bias):
        xw = _matmul(h, w)
        agg = jnp.zeros_like(xw).at[dst].add(xw[src] * norm_e)
        agg = agg + xw * dinv2
        return jnp.maximum(agg + bias, 0.0)

    xp = jnp.pad(x, ((0, 0), (0, 4)))
    w1p = jnp.pad(W1, ((0, 4), (0, 4)))
    b1p = jnp.pad(b1, (0, 4))
    w2p = jnp.pad(W2, ((0, 4), (0, 8)))
    b2p = jnp.pad(b2, (0, 8))
    w3p = jnp.pad(W3, ((0, 8), (0, 4)))
    b3p = jnp.pad(b3, (0, 4))
    wgp = jnp.pad(Wg, ((0, 4), (0, 0)))

    h1 = conv(xp, w1p, b1p)
    h2 = conv(h1, w2p, b2p)
    h3 = conv(h2, w3p, b3p)

    pooled = jax.ops.segment_max(h3, batch, num_segments=bsz)
    return _head(pooled, wgp, bg, feature, Wf1, bf1, Wf2, bf2)


# speedup vs baseline: 2.1261x; 2.0362x over previous
import functools

import jax
import jax.numpy as jnp
from jax.experimental import pallas as pl

_N_BLOCK = 2000


def _mm0_body(x_ref, w_ref, dinv_ref, o_ref):
    o_ref[...] = jnp.dot(x_ref[...], w_ref[...],
                         preferred_element_type=jnp.float32) * dinv_ref[...]


def _mm_fused_body(s_ref, ys_ref, dinv_ref, b_ref, w_ref, o_ref):
    h = jnp.maximum((s_ref[...] + ys_ref[...]) * dinv_ref[...] + b_ref[...],
                    0.0)
    o_ref[...] = jnp.dot(h, w_ref[...],
                         preferred_element_type=jnp.float32) * dinv_ref[...]


def _matmul0(x, w, dinv):
    n, d = x.shape
    do = w.shape[1]
    return pl.pallas_call(
        _mm0_body,
        grid=(n // _N_BLOCK,),
        in_specs=[
            pl.BlockSpec((_N_BLOCK, d), lambda i: (i, 0)),
            pl.BlockSpec((d, do), lambda i: (0, 0)),
            pl.BlockSpec((_N_BLOCK, 1), lambda i: (i, 0)),
        ],
        out_specs=pl.BlockSpec((_N_BLOCK, do), lambda i: (i, 0)),
        out_shape=jax.ShapeDtypeStruct((n, do), jnp.float32),
    )(x, w, dinv)


def _matmul_fused(s, ys, dinv, b, w):
    n, d = s.shape
    do = w.shape[1]
    return pl.pallas_call(
        _mm_fused_body,
        grid=(n // _N_BLOCK,),
        in_specs=[
            pl.BlockSpec((_N_BLOCK, d), lambda i: (i, 0)),
            pl.BlockSpec((_N_BLOCK, d), lambda i: (i, 0)),
            pl.BlockSpec((_N_BLOCK, 1), lambda i: (i, 0)),
            pl.BlockSpec((1, d), lambda i: (0, 0)),
            pl.BlockSpec((d, do), lambda i: (0, 0)),
        ],
        out_specs=pl.BlockSpec((_N_BLOCK, do), lambda i: (i, 0)),
        out_shape=jax.ShapeDtypeStruct((n, do), jnp.float32),
    )(s, ys, dinv, b, w)


def _head_body(x1_ref, wg_ref, bg_ref, ft_ref, wf1_ref, bf1_ref,
               wf2_ref, bf2_ref, o_ref):
    g = jnp.maximum(
        jnp.dot(x1_ref[...], wg_ref[...],
                preferred_element_type=jnp.float32) + bg_ref[...], 0.0)
    h = jnp.maximum(
        jnp.dot(ft_ref[...], wf1_ref[...],
                preferred_element_type=jnp.float32) + bf1_ref[...], 0.0)
    o_ref[...] = g + jnp.dot(h, wf2_ref[...],
                             preferred_element_type=jnp.float32) + bf2_ref[...]


def _head(x1, wg, bg, feature, wf1, bf1, wf2, bf2):
    b = x1.shape[0]
    return pl.pallas_call(
        _head_body,
        out_shape=jax.ShapeDtypeStruct((b, 1), jnp.float32),
    )(x1, wg, bg.reshape(1, 1), feature, wf1, bf1.reshape(1, 128),
      wf2, bf2.reshape(1, 1))


def kernel(x, edge_index, batch, feature, W1, b1, W2, b2, W3, b3,
           Wg, bg, Wf1, bf1, Wf2, bf2):
    n = x.shape[0]
    bsz = feature.shape[0]
    src = edge_index[0]
    dst = edge_index[1]

    deg = jnp.zeros((n,), jnp.float32).at[dst].add(1.0) + 1.0
    dinv = jax.lax.rsqrt(deg)[:, None]

    xp = jnp.pad(x, ((0, 0), (0, 4)))
    w1p = jnp.pad(W1, ((0, 4), (0, 4)))
    b1p = jnp.pad(b1, (0, 4)).reshape(1, -1)
    w2p = jnp.pad(W2, ((0, 4), (0, 8)))
    b2p = jnp.pad(b2, (0, 8)).reshape(1, -1)
    w3p = jnp.pad(W3, ((0, 8), (0, 4)))
    b3p = jnp.pad(b3, (0, 4)).reshape(1, -1)
    wgp = jnp.pad(Wg, ((0, 4), (0, 0)))

    ys1 = _matmul0(xp, w1p, dinv)
    s1 = jnp.zeros_like(ys1).at[dst].add(ys1[src])
    ys2 = _matmul_fused(s1, ys1, dinv, b1p, w2p)
    s2 = jnp.zeros_like(ys2).at[dst].add(ys2[src])
    ys3 = _matmul_fused(s2, ys2, dinv, b2p, w3p)
    s3 = jnp.zeros_like(ys3).at[dst].add(ys3[src])
    h3 = jnp.maximum((s3 + ys3) * dinv + b3p, 0.0)

    pooled = jax.ops.segment_max(h3, batch, num_segments=bsz)
    return _head(pooled, wgp, bg, feature, Wf1, bf1, Wf2, bf2)
